# baseline (speedup 1.0000x reference)
import functools
import math

import jax
import jax.numpy as jnp
from jax import lax
from jax.experimental import pallas as pl
from jax.experimental.pallas import tpu as pltpu
from jax.experimental.pallas import tpu_sc as plsc

_VD = 16
_D = 16
_W = 15
_L = 256
_NW = 32
_NPAIR = _W * _W
_PR = 512


def _x_kernel(hva_ref, hvb_ref, mwt_ref, out_ref):
    f32 = jnp.float32
    s1 = hva_ref[...] + hvb_ref[...]
    nxt = jnp.concatenate([s1[1:_PR], jnp.zeros((1, 128), f32)], axis=0)
    sh = jnp.concatenate([s1[:, 16:128], nxt[:, 0:16]], axis=1)
    s4 = s1 + sh
    s3 = jnp.concatenate(
        [s4[0:_PR - 2] + s4[2:_PR], jnp.zeros((2, 128), f32)], axis=0)
    s2 = jnp.concatenate(
        [s3[0:_PR - 32] + s3[32:_PR], jnp.zeros((32, 128), f32)], axis=0)
    win = s2 * (1.0 / 16.0)
    mwt = mwt_ref[...]
    su = lax.broadcasted_iota(jnp.int32, (128, _VD), 0)
    sc_ = lax.broadcasted_iota(jnp.int32, (128, _VD), 1)
    sel = ((su % _VD) == sc_).astype(f32)
    tile8 = jnp.dot(jnp.dot(sel, mwt, preferred_element_type=f32),
                    sel.T, preferred_element_type=f32)
    blk = (lax.broadcasted_iota(jnp.int32, (128, 128), 0) // _VD ==
           lax.broadcasted_iota(jnp.int32, (128, 128), 1) // _VD)
    g8 = tile8 * blk.astype(f32)
    out_ref[0] = jnp.dot(win, g8, preferred_element_type=f32)


def _sc_body(ct_hbm, pt_hbm, x_hbm, out_hbm, ct_v, pt_v, e12_v,
             xa_v, xb_v, ob_v, sem_a, sem_b):
    core = lax.axis_index("c")
    sub = lax.axis_index("s")
    wid = sub * 2 + core

    pltpu.sync_copy(ct_hbm, ct_v)
    pltpu.sync_copy(pt_hbm, pt_v)

    def x_src(pair):
        k1 = pair // _W
        k2 = pair - k1 * _W
        return x_hbm.at[k1, pl.ds(k2 * 32, 32)]

    def do_pair(pair, x_v, k3lo, k3hi, obase):
        k1 = pair // _W
        k2 = pair - k1 * _W

        @pl.loop(0, _VD)
        def _e12(j):
            e12_v[pl.ds(j * _VD, _VD)] = (
                ct_v[pl.ds(k1 * _L + j * _VD, _VD)]
                * ct_v[pl.ds(k2 * _L + j * _VD, _VD)]
                * pt_v[pl.ds(j * _VD, _VD)])

        @pl.loop(k3lo, k3hi)
        def _k3(k3):
            e123 = [e12_v[pl.ds(j * _VD, _VD)]
                    * ct_v[pl.ds(k3 * _L + j * _VD, _VD)]
                    for j in range(_VD)]
            row0 = obase + (k3 - k3lo) * _W

            @plsc.parallel_loop(0, _W, unroll=4)
            def _k4(k4):
                w = k3 * _VD + k4
                xrow = x_v[w // 8, pl.ds((w % 8) * _VD, _VD)]
                cbase = k4 * _L
                accs = [None] * 4
                for j in range(_VD):
                    term = ((xrow[j] * e123[j])
                            * ct_v[pl.ds(cbase + j * _VD, _VD)])
                    c = j % 4
                    accs[c] = term if accs[c] is None else accs[c] + term
                ob_v[row0 + k4] = (
                    (accs[0] + accs[1]) + (accs[2] + accs[3]))

    xbufs = (xa_v, xb_v)
    sems = (sem_a, sem_b)

    segs = [(0, 0, _W, 0, None), (1, 0, _W, 225, None), (2, 0, 10, 450, 0),
            (2, 10, _W, 0, None), (3, 0, _W, 75, None), (4, 0, _W, 300, None),
            (5, 0, 5, 525, 1),
            (5, 5, _W, 0, None), (6, 0, _W, 150, None), (7, 0, _W, 375, 2)]

    @pl.when(wid < 28)
    def _main():
        base = wid * 8
        pltpu.async_copy(x_src(base), xa_v, sem_a)
        cur_t = [-1]
        cur_buf = [xa_v]
        for (t, k3lo, k3hi, obase, drain_c) in segs:
            if t != cur_t[0]:
                cur_t[0] = t
                cur_buf[0] = xbufs[t % 2]
                pltpu.make_async_copy(x_src(base + t), xbufs[t % 2],
                                      sems[t % 2]).wait()
                if t < 7:
                    pltpu.async_copy(x_src(base + t + 1), xbufs[(t + 1) % 2],
                                     sems[(t + 1) % 2])
            do_pair(base + t, cur_buf[0], k3lo, k3hi, obase)
            if drain_c is not None:
                pltpu.sync_copy(
                    ob_v,
                    out_hbm.at[pl.ds(wid * 1800 + drain_c * 600, 600)])

    @pl.when(wid == 28)
    def _last():
        pltpu.async_copy(x_src(224), xa_v, sem_a)
        pltpu.make_async_copy(x_src(224), xa_v, sem_a).wait()
        do_pair(224, xa_v, 0, _W, 0)
        pltpu.sync_copy(ob_v.at[pl.ds(0, 225)],
                        out_hbm.at[pl.ds(224 * 225, 225)])


@jax.jit
def kernel(hypervol, M_w, P):
    k_idx = jnp.arange(_D, dtype=jnp.float32)[:, None, None]
    i_idx = jnp.arange(_VD, dtype=jnp.float32)[None, None, :]
    j_idx = jnp.arange(_VD, dtype=jnp.float32)[None, :, None]
    ct = jnp.cos((2.0 * math.pi) * k_idx / (i_idx * _VD + j_idx + 2.0))
    ct_flat = ct.reshape(-1)
    pt_flat = P.T.reshape(-1)

    hv_p = hypervol.reshape(_D * _PR, 128)

    x_p = pl.pallas_call(
        _x_kernel,
        grid=(_W,),
        in_specs=[
            pl.BlockSpec((_PR, 128), lambda i: (i, 0)),
            pl.BlockSpec((_PR, 128), lambda i: (i + 1, 0)),
            pl.BlockSpec((_VD, _VD), lambda i: (0, 0)),
        ],
        out_specs=pl.BlockSpec((1, _PR, 128), lambda i: (i, 0, 0)),
        out_shape=jax.ShapeDtypeStruct((_W, _PR, 128), jnp.float32),
    )(hv_p, hv_p, M_w.T)

    sc_kernel = pl.kernel(
        _sc_body,
        out_type=jax.ShapeDtypeStruct((_W ** 4, _VD), jnp.float32),
        mesh=plsc.VectorSubcoreMesh(core_axis_name="c", subcore_axis_name="s"),
        scratch_types=[
            pltpu.VMEM((_D * _L,), jnp.float32),
            pltpu.VMEM((_L,), jnp.float32),
            pltpu.VMEM((_L,), jnp.float32),
            pltpu.VMEM((32, 128), jnp.float32),
            pltpu.VMEM((32, 128), jnp.float32),
            pltpu.VMEM((600, _VD), jnp.float32),
            pltpu.SemaphoreType.DMA,
            pltpu.SemaphoreType.DMA,
        ],
    )
    return sc_kernel(ct_flat, pt_flat, x_p)


# speedup vs baseline: 1.0651x; 1.0651x over previous
import math

import jax
import jax.numpy as jnp
from jax import lax
from jax.experimental import pallas as pl
from jax.experimental.pallas import tpu as pltpu
from jax.experimental.pallas import tpu_sc as plsc

_VD = 16
_D = 16
_W = 15
_L = 256
_NW = 32
_NPAIR = _W * _W
_PR = 512


def _x_kernel(hva_ref, hvb_ref, mwt_ref, out_ref):
    f32 = jnp.float32
    s1 = hva_ref[...] + hvb_ref[...]
    nxt = jnp.concatenate([s1[1:_PR], jnp.zeros((1, 128), f32)], axis=0)
    sh = jnp.concatenate([s1[:, 16:128], nxt[:, 0:16]], axis=1)
    s4 = s1 + sh
    s3 = jnp.concatenate(
        [s4[0:_PR - 2] + s4[2:_PR], jnp.zeros((2, 128), f32)], axis=0)
    s2 = jnp.concatenate(
        [s3[0:_PR - 32] + s3[32:_PR], jnp.zeros((32, 128), f32)], axis=0)
    win = s2 * (1.0 / 16.0)
    mwt = mwt_ref[...]
    su = lax.broadcasted_iota(jnp.int32, (128, _VD), 0)
    sc_ = lax.broadcasted_iota(jnp.int32, (128, _VD), 1)
    sel = ((su % _VD) == sc_).astype(f32)
    tile8 = jnp.dot(jnp.dot(sel, mwt, preferred_element_type=f32),
                    sel.T, preferred_element_type=f32)
    blk = (lax.broadcasted_iota(jnp.int32, (128, 128), 0) // _VD ==
           lax.broadcasted_iota(jnp.int32, (128, 128), 1) // _VD)
    g8 = tile8 * blk.astype(f32)
    out_ref[0] = jnp.dot(win, g8, preferred_element_type=f32)


def _sc_body(ct_hbm, pt_hbm, x_hbm, out_hbm, ct_v, pt_v, e12_v,
             xa_v, xb_v, ob_v, sem_a, sem_b):
    core = lax.axis_index("c")
    sub = lax.axis_index("s")
    wid = sub * 2 + core

    pltpu.sync_copy(ct_hbm, ct_v)
    pltpu.sync_copy(pt_hbm, pt_v)

    def x_src(pair):
        k1 = pair // _W
        k2 = pair - k1 * _W
        return x_hbm.at[k1, pl.ds(k2 * 32, 32)]

    def do_pair(pair, x_v, k3lo, k3hi, obase):
        k1 = pair // _W
        k2 = pair - k1 * _W

        @pl.loop(0, _VD)
        def _e12(j):
            e12_v[pl.ds(j * _VD, _VD)] = (
                ct_v[pl.ds(k1 * _L + j * _VD, _VD)]
                * ct_v[pl.ds(k2 * _L + j * _VD, _VD)]
                * pt_v[pl.ds(j * _VD, _VD)])

        @pl.loop(k3lo, k3hi)
        def _k3(k3):
            e123 = [e12_v[pl.ds(j * _VD, _VD)]
                    * ct_v[pl.ds(k3 * _L + j * _VD, _VD)]
                    for j in range(_VD)]
            row0 = obase + (k3 - k3lo) * _W

            @plsc.parallel_loop(0, _W, unroll=3)
            def _k4(k4):
                w = k3 * _VD + k4
                xrow = x_v[w // 8, pl.ds((w % 8) * _VD, _VD)]
                cbase = k4 * _L
                accs = [None] * 4
                for j in range(_VD):
                    term = ((xrow[j] * e123[j])
                            * ct_v[pl.ds(cbase + j * _VD, _VD)])
                    c = j % 4
                    accs[c] = term if accs[c] is None else accs[c] + term
                ob_v[row0 + k4] = (
                    (accs[0] + accs[1]) + (accs[2] + accs[3]))

    xbufs = (xa_v, xb_v)
    sems = (sem_a, sem_b)

    segs = [(0, 0, _W, 0, None), (1, 0, _W, 225, None), (2, 0, 10, 450, 0),
            (2, 10, _W, 0, None), (3, 0, _W, 75, None), (4, 0, _W, 300, None),
            (5, 0, 5, 525, 1),
            (5, 5, _W, 0, None), (6, 0, _W, 150, None), (7, 0, _W, 375, 2)]

    @pl.when(wid < 28)
    def _main():
        base = wid * 8
        pltpu.async_copy(x_src(base), xa_v, sem_a)
        cur_t = [-1]
        cur_buf = [xa_v]
        for (t, k3lo, k3hi, obase, drain_c) in segs:
            if t != cur_t[0]:
                cur_t[0] = t
                cur_buf[0] = xbufs[t % 2]
                pltpu.make_async_copy(x_src(base + t), xbufs[t % 2],
                                      sems[t % 2]).wait()
                if t < 7:
                    pltpu.async_copy(x_src(base + t + 1), xbufs[(t + 1) % 2],
                                     sems[(t + 1) % 2])
            do_pair(base + t, cur_buf[0], k3lo, k3hi, obase)
            if drain_c is not None:
                pltpu.sync_copy(
                    ob_v,
                    out_hbm.at[pl.ds(wid * 1800 + drain_c * 600, 600)])

    @pl.when(wid == 28)
    def _last():
        pltpu.async_copy(x_src(224), xa_v, sem_a)
        pltpu.make_async_copy(x_src(224), xa_v, sem_a).wait()
        do_pair(224, xa_v, 0, _W, 0)
        pltpu.sync_copy(ob_v.at[pl.ds(0, 225)],
                        out_hbm.at[pl.ds(224 * 225, 225)])


@jax.jit
def kernel(hypervol, M_w, P):
    k_idx = jnp.arange(_D, dtype=jnp.float32)[:, None, None]
    i_idx = jnp.arange(_VD, dtype=jnp.float32)[None, None, :]
    j_idx = jnp.arange(_VD, dtype=jnp.float32)[None, :, None]
    ct = jnp.cos((2.0 * math.pi) * k_idx / (i_idx * _VD + j_idx + 2.0))
    ct_flat = ct.reshape(-1)
    pt_flat = P.T.reshape(-1)

    hv_p = hypervol.reshape(_D * _PR, 128)

    x_p = pl.pallas_call(
        _x_kernel,
        grid=(_W,),
        in_specs=[
            pl.BlockSpec((_PR, 128), lambda i: (i, 0)),
            pl.BlockSpec((_PR, 128), lambda i: (i + 1, 0)),
            pl.BlockSpec((_VD, _VD), lambda i: (0, 0)),
        ],
        out_specs=pl.BlockSpec((1, _PR, 128), lambda i: (i, 0, 0)),
        out_shape=jax.ShapeDtypeStruct((_W, _PR, 128), jnp.float32),
    )(hv_p, hv_p, M_w.T)

    sc_kernel = pl.kernel(
        _sc_body,
        out_type=jax.ShapeDtypeStruct((_W ** 4, _VD), jnp.float32),
        mesh=plsc.VectorSubcoreMesh(core_axis_name="c", subcore_axis_name="s"),
        scratch_types=[
            pltpu.VMEM((_D * _L,), jnp.float32),
            pltpu.VMEM((_L,), jnp.float32),
            pltpu.VMEM((_L,), jnp.float32),
            pltpu.VMEM((32, 128), jnp.float32),
            pltpu.VMEM((32, 128), jnp.float32),
            pltpu.VMEM((600, _VD), jnp.float32),
            pltpu.SemaphoreType.DMA,
            pltpu.SemaphoreType.DMA,
        ],
    )
    return sc_kernel(ct_flat, pt_flat, x_p)
